# baseline (speedup 1.0000x reference)
import functools

import jax
import jax.numpy as jnp
from jax import lax
from jax.experimental import pallas as pl
from jax.experimental.pallas import tpu as pltpu
from jax.experimental.pallas import tpu_sc as plsc

_N = 5000
_NPAD = 5120
_NSC = 512
_NTC = _N - _NSC
_BI = 1496
_THRESHOLD = 0.8


def _tc_kernel(rows_ref, cols_ref, out_ref):
    x1i = rows_ref[:, 0:1]
    y1i = rows_ref[:, 1:2]
    x2i = rows_ref[:, 2:3]
    y2i = rows_ref[:, 3:4]
    x1j = cols_ref[0:1, :]
    y1j = cols_ref[1:2, :]
    x2j = cols_ref[2:3, :]
    y2j = cols_ref[3:4, :]
    aj = (x2j - x1j) * (y2j - y1j)
    ajb = jnp.broadcast_to(aj, (_BI, _NPAD))
    z = jnp.zeros((_BI, _NPAD), jnp.float32)
    contrib = jnp.where(
        (x1j >= x1i) & (y1j >= y1i),
        jnp.where((x2j <= x2i) & (y2j <= y2i), ajb, z),
        z,
    )
    s = jnp.sum(contrib, axis=1, keepdims=True)
    ai = (x2i - x1i) * (y2i - y1i)
    keep = (s - ai) <= _THRESHOLD * (ai + 1e-9)
    out_ref[:, :] = rows_ref[:, :] * keep.astype(jnp.float32)


@functools.partial(
    pl.kernel,
    out_type=jax.ShapeDtypeStruct((_NSC,), jnp.float32),
    mesh=plsc.VectorSubcoreMesh(core_axis_name="c", subcore_axis_name="s"),
    scratch_types=[
        pltpu.VMEM((8, _NPAD), jnp.float32),
        pltpu.VMEM((4, 16), jnp.float32),
        pltpu.VMEM((16,), jnp.float32),
    ],
)
def _sc_kernel(cols_hbm, rows_flat_hbm, keep_hbm, cols_v, rows_v, keep_v):
    wid = lax.axis_index("s") * 2 + lax.axis_index("c")
    base = wid * 16
    pltpu.sync_copy(cols_hbm, cols_v)
    for k in range(4):
        pltpu.sync_copy(
            rows_flat_hbm.at[pl.ds(k * _NSC + base, 16)], rows_v.at[k]
        )
    x1i_v = rows_v[0, :]
    y1i_v = rows_v[1, :]
    x2i_v = rows_v[2, :]
    y2i_v = rows_v[3, :]
    z16 = jnp.zeros((16,), jnp.float32)

    def chunk_body(ci, s16):
        c0 = cols_v[0, pl.ds(ci * 16, 16)]
        c1 = cols_v[1, pl.ds(ci * 16, 16)]
        c2 = cols_v[2, pl.ds(ci * 16, 16)]
        c3 = cols_v[3, pl.ds(ci * 16, 16)]
        for k in range(16):
            x1j = jnp.full((16,), c0[k])
            y1j = jnp.full((16,), c1[k])
            x2j = jnp.full((16,), c2[k])
            y2j = jnp.full((16,), c3[k])
            aj = (x2j - x1j) * (y2j - y1j)
            contrib = jnp.where(
                (x1j >= x1i_v) & (y1j >= y1i_v),
                jnp.where((x2j <= x2i_v) & (y2j <= y2i_v), aj, z16),
                z16,
            )
            s16 = s16 + contrib
        return s16

    s16 = lax.fori_loop(0, _NPAD // 16, chunk_body, z16)
    ai_v = (x2i_v - x1i_v) * (y2i_v - y1i_v)
    keep = (s16 - ai_v) <= _THRESHOLD * (ai_v + 1e-9)
    keep_v[:] = jnp.where(keep, jnp.ones((16,), jnp.float32), z16)
    pltpu.sync_copy(keep_v, keep_hbm.at[pl.ds(wid * 16, 16)])


@jax.jit
def kernel(boxes):
    cols = jnp.zeros((8, _NPAD), jnp.float32).at[:4, :_N].set(boxes.T)
    out_tc = pl.pallas_call(
        _tc_kernel,
        grid=(_NTC // _BI,),
        in_specs=[
            pl.BlockSpec((_BI, 4), lambda i: (i, 0)),
            pl.BlockSpec((8, _NPAD), lambda i: (0, 0)),
        ],
        out_specs=pl.BlockSpec((_BI, 4), lambda i: (i, 0)),
        out_shape=jax.ShapeDtypeStruct((_NTC, 4), jnp.float32),
        compiler_params=pltpu.CompilerParams(
            dimension_semantics=("arbitrary",),
        ),
    )(boxes[:_NTC], cols)
    rows_flat = cols[0:4, _NTC:_N].reshape(4 * _NSC)
    keep_sc = _sc_kernel(cols, rows_flat)
    out_sc = boxes[_NTC:] * keep_sc[:, None]
    return jnp.concatenate([out_tc, out_sc], axis=0)


# speedup vs baseline: 2.2785x; 2.2785x over previous
import jax
import jax.numpy as jnp
from jax.experimental import pallas as pl
from jax.experimental.pallas import tpu as pltpu

_N = 5000
_NPAD = 5120
_BI = 1000
_THRESHOLD = 0.8


def _contain_kernel(rows_ref, full_ref, out_ref, cols_ref):
    @pl.when(pl.program_id(0) == 0)
    def _build_cols():
        cols_ref[:, :] = jnp.zeros((8, _NPAD), jnp.float32)
        cols_ref[0:4, 0:_N] = jnp.transpose(full_ref[:, :])

    x1i = rows_ref[:, 0:1]
    y1i = rows_ref[:, 1:2]
    x2i = rows_ref[:, 2:3]
    y2i = rows_ref[:, 3:4]
    x1j = cols_ref[0:1, :]
    y1j = cols_ref[1:2, :]
    x2j = cols_ref[2:3, :]
    y2j = cols_ref[3:4, :]
    aj = (x2j - x1j) * (y2j - y1j)
    ajb = jnp.broadcast_to(aj, (_BI, _NPAD))
    z = jnp.zeros((_BI, _NPAD), jnp.float32)
    contrib = jnp.where(
        (x1j >= x1i) & (y1j >= y1i),
        jnp.where((x2j <= x2i) & (y2j <= y2i), ajb, z),
        z,
    )
    s = jnp.sum(contrib, axis=1, keepdims=True)
    ai = (x2i - x1i) * (y2i - y1i)
    keep = (s - ai) <= _THRESHOLD * (ai + 1e-9)
    out_ref[:, :] = rows_ref[:, :] * keep.astype(jnp.float32)


@jax.jit
def kernel(boxes):
    return pl.pallas_call(
        _contain_kernel,
        grid=(_N // _BI,),
        in_specs=[
            pl.BlockSpec((_BI, 4), lambda i: (i, 0)),
            pl.BlockSpec((_N, 4), lambda i: (0, 0)),
        ],
        out_specs=pl.BlockSpec((_BI, 4), lambda i: (i, 0)),
        out_shape=jax.ShapeDtypeStruct((_N, 4), jnp.float32),
        scratch_shapes=[pltpu.VMEM((8, _NPAD), jnp.float32)],
        compiler_params=pltpu.CompilerParams(
            dimension_semantics=("arbitrary",),
        ),
    )(boxes, boxes)
